# Initial kernel scaffold; baseline (speedup 1.0000x reference)
import jax
import jax.numpy as jnp
from jax.experimental import pallas as pl


def kernel(x, router_weight, eg_w, eg_s, eu_w, eu_s, ed_w, ed_s, sg_w, sg_s, su_w, su_s, sd_w, sd_s):
    raise NotImplementedError("write your pallas kernel here")


# speedup vs baseline: 2.9737x; 2.9737x over previous
import functools

import jax
import jax.numpy as jnp
from jax import lax
from jax.experimental import pallas as pl
from jax.experimental.pallas import tpu as pltpu
from jax.experimental.pallas import tpu_sc as plsc

T = 2048
D = 768
I = 2048
E = 64
CAP = 128
NW = 32
TPW = T // NW
XE_ROWS = E * CAP + CAP
IC = 2


def _router_body(x_ref, rw_ref, dd0_ref, dd1_ref, dc0_ref, dc1_ref,
                 g0_ref, g1_ref):
    x = x_ref[...]
    rw = rw_ref[...]
    logits = lax.dot_general(
        x.astype(jnp.bfloat16), rw.astype(jnp.bfloat16),
        (((1,), (1,)), ((), ())),
        preferred_element_type=jnp.float32)
    m = jnp.max(logits, axis=1, keepdims=True)
    ex = jnp.exp(logits - m)
    probs = ex / jnp.sum(ex, axis=1, keepdims=True)

    cols = lax.broadcasted_iota(jnp.int32, (T, E), 1)
    big = jnp.int32(E)
    v0 = jnp.max(probs, axis=1, keepdims=True)
    i0 = jnp.min(jnp.where(probs == v0, cols, big), axis=1, keepdims=True)
    masked = jnp.where(cols == i0, -jnp.inf, probs)
    v1 = jnp.max(masked, axis=1, keepdims=True)
    i1 = jnp.min(jnp.where(masked == v1, cols, big), axis=1, keepdims=True)

    ssum = v0 + v1
    gate0 = v0 / ssum
    gate1 = v1 / ssum

    oh0 = (cols == i0).astype(jnp.float32)
    oh1 = (cols == i1).astype(jnp.float32)
    ohs = oh0 + oh1
    c = ohs
    s = 1
    while s < T:
        zero = jnp.zeros((s, E), jnp.float32)
        c = c + jnp.concatenate([zero, c[:-s, :]], axis=0)
        s *= 2
    c_excl = c - ohs
    pos0 = jnp.sum(c_excl * oh0, axis=1, keepdims=True).astype(jnp.int32)
    pos1 = jnp.sum(c_excl * oh1, axis=1, keepdims=True).astype(jnp.int32)

    rows = lax.broadcasted_iota(jnp.int32, (T, 1), 0)
    dump = E * CAP + (rows % CAP)
    keep0 = pos0 < CAP
    keep1 = pos1 < CAP
    dd0_ref[...] = jnp.where(keep0, i0 * CAP + pos0, dump)
    dd1_ref[...] = jnp.where(keep1, i1 * CAP + pos1, dump)
    dc0_ref[...] = i0 * CAP + jnp.minimum(pos0, CAP - 1)
    dc1_ref[...] = i1 * CAP + jnp.minimum(pos1, CAP - 1)
    g0_ref[...] = jnp.where(keep0, gate0, 0.0)
    g1_ref[...] = jnp.where(keep1, gate1, 0.0)


def _router(x, router_weight):
    outs = pl.pallas_call(
        _router_body,
        out_shape=[
            jax.ShapeDtypeStruct((T, 1), jnp.int32),
            jax.ShapeDtypeStruct((T, 1), jnp.int32),
            jax.ShapeDtypeStruct((T, 1), jnp.int32),
            jax.ShapeDtypeStruct((T, 1), jnp.int32),
            jax.ShapeDtypeStruct((T, 1), jnp.float32),
            jax.ShapeDtypeStruct((T, 1), jnp.float32),
        ],
    )(x, router_weight)
    return outs


def _dispatch_body(x_hbm, dd0_hbm, dd1_hbm, xe_hbm, i0_v, i1_v, rows_v, sem):
    wid = lax.axis_index("s") * 2 + lax.axis_index("c")
    base = wid * TPW
    pltpu.sync_copy(x_hbm.at[pl.ds(base, TPW)], rows_v)
    pltpu.sync_copy(dd0_hbm.at[pl.ds(base, TPW)], i0_v)
    pltpu.sync_copy(dd1_hbm.at[pl.ds(base, TPW)], i1_v)
    pltpu.async_copy(rows_v, xe_hbm.at[i0_v], sem).wait()
    pltpu.async_copy(rows_v, xe_hbm.at[i1_v], sem).wait()


def _dispatch(x, dd0, dd1):
    mesh = plsc.VectorSubcoreMesh(core_axis_name="c", subcore_axis_name="s")
    fn = functools.partial(
        pl.kernel, mesh=mesh,
        out_type=jax.ShapeDtypeStruct((XE_ROWS, D), jnp.float32),
        scratch_types=[
            pltpu.VMEM((TPW,), jnp.int32),
            pltpu.VMEM((TPW,), jnp.int32),
            pltpu.VMEM((TPW, D), jnp.float32),
            pltpu.SemaphoreType.DMA,
        ],
    )(_dispatch_body)
    return fn(x, dd0, dd1)


def _expert_body(xe_ref, gw_ref, uw_ref, dw_ref, gs_ref, us_ref, ds_ref,
                 ys_ref):
    j = pl.program_id(1)
    h = xe_ref[...].astype(jnp.bfloat16)
    gw = gw_ref[0].astype(jnp.bfloat16)
    uw = uw_ref[0].astype(jnp.bfloat16)
    dn = (((1,), (1,)), ((), ()))
    g = lax.dot_general(h, gw, dn, preferred_element_type=jnp.float32)
    u = lax.dot_general(h, uw, dn, preferred_element_type=jnp.float32)
    g = g * gs_ref[0, 0, 0]
    u = u * us_ref[0, 0, 0]
    g = g / (1.0 + jnp.exp(-g))
    p = (g * u).astype(jnp.bfloat16)
    dw = dw_ref[0].astype(jnp.bfloat16)
    y = lax.dot_general(p, dw, dn, preferred_element_type=jnp.float32)
    y = y * ds_ref[0, 0, 0]

    @pl.when(j == 0)
    def _():
        ys_ref[...] = y

    @pl.when(j != 0)
    def _():
        ys_ref[...] += y


def _experts(xe, eg_w, eg_s, eu_w, eu_s, ed_w, ed_s):
    ichunk = I // IC
    smem1 = pl.BlockSpec((1, 1, 1), lambda e, j: (e, 0, 0),
                         memory_space=pltpu.SMEM)
    return pl.pallas_call(
        _expert_body,
        grid=(E, IC),
        in_specs=[
            pl.BlockSpec((CAP, D), lambda e, j: (e, 0)),
            pl.BlockSpec((1, ichunk, D), lambda e, j: (e, j, 0)),
            pl.BlockSpec((1, ichunk, D), lambda e, j: (e, j, 0)),
            pl.BlockSpec((1, D, ichunk), lambda e, j: (e, 0, j)),
            smem1, smem1, smem1,
        ],
        out_specs=pl.BlockSpec((CAP, D), lambda e, j: (e, 0)),
        out_shape=jax.ShapeDtypeStruct((E * CAP, D), jnp.float32),
        compiler_params=pltpu.CompilerParams(
            dimension_semantics=("arbitrary", "arbitrary")),
    )(xe[:E * CAP], eg_w, eu_w, ed_w,
      eg_s.reshape(E, 1, 1), eu_s.reshape(E, 1, 1), ed_s.reshape(E, 1, 1))


def _combine_body(ys_hbm, dc0_hbm, dc1_hbm, y0_hbm, y1_hbm,
                  i_v, rows_v, sem):
    wid = lax.axis_index("s") * 2 + lax.axis_index("c")
    base = wid * TPW
    pltpu.sync_copy(dc0_hbm.at[pl.ds(base, TPW)], i_v)
    pltpu.async_copy(ys_hbm.at[i_v], rows_v, sem).wait()
    pltpu.sync_copy(rows_v, y0_hbm.at[pl.ds(base, TPW)])
    pltpu.sync_copy(dc1_hbm.at[pl.ds(base, TPW)], i_v)
    pltpu.async_copy(ys_hbm.at[i_v], rows_v, sem).wait()
    pltpu.sync_copy(rows_v, y1_hbm.at[pl.ds(base, TPW)])


def _combine(ys, dc0, dc1):
    mesh = plsc.VectorSubcoreMesh(core_axis_name="c", subcore_axis_name="s")
    fn = functools.partial(
        pl.kernel, mesh=mesh,
        out_type=[
            jax.ShapeDtypeStruct((T, D), jnp.float32),
            jax.ShapeDtypeStruct((T, D), jnp.float32),
        ],
        scratch_types=[
            pltpu.VMEM((TPW,), jnp.int32),
            pltpu.VMEM((TPW, D), jnp.float32),
            pltpu.SemaphoreType.DMA,
        ],
    )(_combine_body)
    return fn(ys, dc0, dc1)


TT = 256


def _shared_body(x_ref, gw_ref, uw_ref, dw_ref, gs_ref, us_ref, ds_ref,
                 y0_ref, y1_ref, g0_ref, g1_ref, out_ref):
    j = pl.program_id(1)
    xb = x_ref[...].astype(jnp.bfloat16)
    gw = gw_ref[...].astype(jnp.bfloat16)
    uw = uw_ref[...].astype(jnp.bfloat16)
    dn = (((1,), (1,)), ((), ()))
    g = lax.dot_general(xb, gw, dn, preferred_element_type=jnp.float32)
    u = lax.dot_general(xb, uw, dn, preferred_element_type=jnp.float32)
    g = g * gs_ref[0]
    u = u * us_ref[0]
    g = g / (1.0 + jnp.exp(-g))
    p = (g * u).astype(jnp.bfloat16)
    dw = dw_ref[...].astype(jnp.bfloat16)
    y = lax.dot_general(p, dw, dn, preferred_element_type=jnp.float32)
    y = y * ds_ref[0]

    @pl.when(j == 0)
    def _():
        out_ref[...] = y + g0_ref[...] * y0_ref[...] + g1_ref[...] * y1_ref[...]

    @pl.when(j != 0)
    def _():
        out_ref[...] += y


def _shared(x, sg_w, sg_s, su_w, su_s, sd_w, sd_s, y0, y1, g0, g1):
    ichunk = I // IC
    smem1 = pl.BlockSpec((1,), lambda i, j: (0,), memory_space=pltpu.SMEM)
    tok = pl.BlockSpec((TT, D), lambda i, j: (i, 0))
    gsp = pl.BlockSpec((TT, 1), lambda i, j: (i, 0))
    return pl.pallas_call(
        _shared_body,
        grid=(T // TT, IC),
        in_specs=[
            tok,
            pl.BlockSpec((ichunk, D), lambda i, j: (j, 0)),
            pl.BlockSpec((ichunk, D), lambda i, j: (j, 0)),
            pl.BlockSpec((D, ichunk), lambda i, j: (0, j)),
            smem1, smem1, smem1,
            tok, tok, gsp, gsp,
        ],
        out_specs=tok,
        out_shape=jax.ShapeDtypeStruct((T, D), jnp.float32),
        compiler_params=pltpu.CompilerParams(
            dimension_semantics=("arbitrary", "arbitrary")),
    )(x, sg_w, su_w, sd_w, sg_s, su_s, sd_s, y0, y1, g0, g1)


def kernel(x, router_weight, eg_w, eg_s, eu_w, eu_s, ed_w, ed_s,
           sg_w, sg_s, su_w, su_s, sd_w, sd_s):
    dd0, dd1, dc0, dc1, g0, g1 = _router(x, router_weight)
    xe = _dispatch(x, dd0.reshape(T), dd1.reshape(T))
    ys = _experts(xe, eg_w, eg_s, eu_w, eu_s, ed_w, ed_s)
    y0, y1 = _combine(ys, dc0.reshape(T), dc1.reshape(T))
    return _shared(x, sg_w, sg_s, su_w, su_s, sd_w, sd_s, y0, y1, g0, g1)


# speedup vs baseline: 3.3440x; 1.1245x over previous
import functools

import jax
import jax.numpy as jnp
from jax import lax
from jax.experimental import pallas as pl
from jax.experimental.pallas import tpu as pltpu
from jax.experimental.pallas import tpu_sc as plsc

T = 2048
D = 768
I = 2048
E = 64
CAP = 128
NW = 32
TPW = T // NW
XE_ROWS = E * CAP + CAP
IC = 1


def _router_body(x_ref, rw_ref, dd0_ref, dd1_ref, dc0_ref, dc1_ref,
                 g0_ref, g1_ref):
    x = x_ref[...]
    rw = rw_ref[...]
    logits = lax.dot_general(
        x.astype(jnp.bfloat16), rw.astype(jnp.bfloat16),
        (((1,), (1,)), ((), ())),
        preferred_element_type=jnp.float32)
    m = jnp.max(logits, axis=1, keepdims=True)
    ex = jnp.exp(logits - m)
    probs = ex / jnp.sum(ex, axis=1, keepdims=True)

    cols = lax.broadcasted_iota(jnp.int32, (T, E), 1)
    big = jnp.int32(E)
    v0 = jnp.max(probs, axis=1, keepdims=True)
    i0 = jnp.min(jnp.where(probs == v0, cols, big), axis=1, keepdims=True)
    masked = jnp.where(cols == i0, -jnp.inf, probs)
    v1 = jnp.max(masked, axis=1, keepdims=True)
    i1 = jnp.min(jnp.where(masked == v1, cols, big), axis=1, keepdims=True)

    ssum = v0 + v1
    gate0 = v0 / ssum
    gate1 = v1 / ssum

    oh0 = (cols == i0).astype(jnp.float32)
    oh1 = (cols == i1).astype(jnp.float32)
    ohs = oh0 + oh1
    c = ohs
    s = 1
    while s < T:
        zero = jnp.zeros((s, E), jnp.float32)
        c = c + jnp.concatenate([zero, c[:-s, :]], axis=0)
        s *= 2
    c_excl = c - ohs
    pos0 = jnp.sum(c_excl * oh0, axis=1, keepdims=True).astype(jnp.int32)
    pos1 = jnp.sum(c_excl * oh1, axis=1, keepdims=True).astype(jnp.int32)

    rows = lax.broadcasted_iota(jnp.int32, (T, 1), 0)
    dump = E * CAP + (rows % CAP)
    keep0 = pos0 < CAP
    keep1 = pos1 < CAP
    dd0_ref[...] = jnp.where(keep0, i0 * CAP + pos0, dump)
    dd1_ref[...] = jnp.where(keep1, i1 * CAP + pos1, dump)
    dc0_ref[...] = i0 * CAP + jnp.minimum(pos0, CAP - 1)
    dc1_ref[...] = i1 * CAP + jnp.minimum(pos1, CAP - 1)
    g0_ref[...] = jnp.where(keep0, gate0, 0.0)
    g1_ref[...] = jnp.where(keep1, gate1, 0.0)


def _router(x, router_weight):
    outs = pl.pallas_call(
        _router_body,
        out_shape=[
            jax.ShapeDtypeStruct((T, 1), jnp.int32),
            jax.ShapeDtypeStruct((T, 1), jnp.int32),
            jax.ShapeDtypeStruct((T, 1), jnp.int32),
            jax.ShapeDtypeStruct((T, 1), jnp.int32),
            jax.ShapeDtypeStruct((T, 1), jnp.float32),
            jax.ShapeDtypeStruct((T, 1), jnp.float32),
        ],
    )(x, router_weight)
    return outs


def _dispatch_body(x_hbm, dd0_hbm, dd1_hbm, xe_hbm, i0_v, i1_v, rows_v, sem):
    wid = lax.axis_index("s") * 2 + lax.axis_index("c")
    base = wid * TPW
    pltpu.sync_copy(x_hbm.at[pl.ds(base, TPW)], rows_v)
    pltpu.sync_copy(dd0_hbm.at[pl.ds(base, TPW)], i0_v)
    pltpu.sync_copy(dd1_hbm.at[pl.ds(base, TPW)], i1_v)
    pltpu.async_copy(rows_v, xe_hbm.at[i0_v], sem).wait()
    pltpu.async_copy(rows_v, xe_hbm.at[i1_v], sem).wait()


def _dispatch(x, dd0, dd1):
    mesh = plsc.VectorSubcoreMesh(core_axis_name="c", subcore_axis_name="s")
    fn = functools.partial(
        pl.kernel, mesh=mesh,
        out_type=jax.ShapeDtypeStruct((XE_ROWS, D), jnp.float32),
        scratch_types=[
            pltpu.VMEM((TPW,), jnp.int32),
            pltpu.VMEM((TPW,), jnp.int32),
            pltpu.VMEM((TPW, D), jnp.float32),
            pltpu.SemaphoreType.DMA,
        ],
    )(_dispatch_body)
    return fn(x, dd0, dd1)


def _expert_body(xe_ref, gw_ref, uw_ref, dw_ref, gs_ref, us_ref, ds_ref,
                 ys_ref):
    h = xe_ref[...].astype(jnp.bfloat16)
    gw = gw_ref[0].astype(jnp.bfloat16)
    uw = uw_ref[0].astype(jnp.bfloat16)
    dn = (((1,), (1,)), ((), ()))
    g = lax.dot_general(h, gw, dn, preferred_element_type=jnp.float32)
    u = lax.dot_general(h, uw, dn, preferred_element_type=jnp.float32)
    g = g * gs_ref[0, 0, 0]
    u = u * us_ref[0, 0, 0]
    g = g / (1.0 + jnp.exp(-g))
    p = (g * u).astype(jnp.bfloat16)
    dw = dw_ref[0].astype(jnp.bfloat16)
    y = lax.dot_general(p, dw, dn, preferred_element_type=jnp.float32)
    ys_ref[...] = y * ds_ref[0, 0, 0]


def _experts(xe, eg_w, eg_s, eu_w, eu_s, ed_w, ed_s):
    smem1 = pl.BlockSpec((1, 1, 1), lambda e: (e, 0, 0),
                         memory_space=pltpu.SMEM)
    return pl.pallas_call(
        _expert_body,
        grid=(E,),
        in_specs=[
            pl.BlockSpec((CAP, D), lambda e: (e, 0)),
            pl.BlockSpec((1, I, D), lambda e: (e, 0, 0)),
            pl.BlockSpec((1, I, D), lambda e: (e, 0, 0)),
            pl.BlockSpec((1, D, I), lambda e: (e, 0, 0)),
            smem1, smem1, smem1,
        ],
        out_specs=pl.BlockSpec((CAP, D), lambda e: (e, 0)),
        out_shape=jax.ShapeDtypeStruct((E * CAP, D), jnp.float32),
        compiler_params=pltpu.CompilerParams(
            dimension_semantics=("arbitrary",)),
    )(xe[:E * CAP], eg_w, eu_w, ed_w,
      eg_s.reshape(E, 1, 1), eu_s.reshape(E, 1, 1), ed_s.reshape(E, 1, 1))


def _combine_body(ys_hbm, dc0_hbm, dc1_hbm, y0_hbm, y1_hbm,
                  i_v, rows_v, sem):
    wid = lax.axis_index("s") * 2 + lax.axis_index("c")
    base = wid * TPW
    pltpu.sync_copy(dc0_hbm.at[pl.ds(base, TPW)], i_v)
    pltpu.async_copy(ys_hbm.at[i_v], rows_v, sem).wait()
    pltpu.sync_copy(rows_v, y0_hbm.at[pl.ds(base, TPW)])
    pltpu.sync_copy(dc1_hbm.at[pl.ds(base, TPW)], i_v)
    pltpu.async_copy(ys_hbm.at[i_v], rows_v, sem).wait()
    pltpu.sync_copy(rows_v, y1_hbm.at[pl.ds(base, TPW)])


def _combine(ys, dc0, dc1):
    mesh = plsc.VectorSubcoreMesh(core_axis_name="c", subcore_axis_name="s")
    fn = functools.partial(
        pl.kernel, mesh=mesh,
        out_type=[
            jax.ShapeDtypeStruct((T, D), jnp.float32),
            jax.ShapeDtypeStruct((T, D), jnp.float32),
        ],
        scratch_types=[
            pltpu.VMEM((TPW,), jnp.int32),
            pltpu.VMEM((TPW, D), jnp.float32),
            pltpu.SemaphoreType.DMA,
        ],
    )(_combine_body)
    return fn(ys, dc0, dc1)


TT = 256


def _shared_body(x_ref, gw_ref, uw_ref, dw_ref, gs_ref, us_ref, ds_ref,
                 y0_ref, y1_ref, g0_ref, g1_ref, out_ref):
    xb = x_ref[...].astype(jnp.bfloat16)
    gw = gw_ref[...].astype(jnp.bfloat16)
    uw = uw_ref[...].astype(jnp.bfloat16)
    dn = (((1,), (1,)), ((), ()))
    g = lax.dot_general(xb, gw, dn, preferred_element_type=jnp.float32)
    u = lax.dot_general(xb, uw, dn, preferred_element_type=jnp.float32)
    g = g * gs_ref[0]
    u = u * us_ref[0]
    g = g / (1.0 + jnp.exp(-g))
    p = (g * u).astype(jnp.bfloat16)
    dw = dw_ref[...].astype(jnp.bfloat16)
    y = lax.dot_general(p, dw, dn, preferred_element_type=jnp.float32)
    y = y * ds_ref[0]
    out_ref[...] = y + g0_ref[...] * y0_ref[...] + g1_ref[...] * y1_ref[...]


def _shared(x, sg_w, sg_s, su_w, su_s, sd_w, sd_s, y0, y1, g0, g1):
    smem1 = pl.BlockSpec((1,), lambda i: (0,), memory_space=pltpu.SMEM)
    tok = pl.BlockSpec((TT, D), lambda i: (i, 0))
    gsp = pl.BlockSpec((TT, 1), lambda i: (i, 0))
    return pl.pallas_call(
        _shared_body,
        grid=(T // TT,),
        in_specs=[
            tok,
            pl.BlockSpec((I, D), lambda i: (0, 0)),
            pl.BlockSpec((I, D), lambda i: (0, 0)),
            pl.BlockSpec((D, I), lambda i: (0, 0)),
            smem1, smem1, smem1,
            tok, tok, gsp, gsp,
        ],
        out_specs=tok,
        out_shape=jax.ShapeDtypeStruct((T, D), jnp.float32),
        compiler_params=pltpu.CompilerParams(
            dimension_semantics=("arbitrary",)),
    )(x, sg_w, su_w, sd_w, sg_s, su_s, sd_s, y0, y1, g0, g1)


def kernel(x, router_weight, eg_w, eg_s, eu_w, eu_s, ed_w, ed_s,
           sg_w, sg_s, su_w, su_s, sd_w, sd_s):
    dd0, dd1, dc0, dc1, g0, g1 = _router(x, router_weight)
    xe = _dispatch(x, dd0.reshape(T), dd1.reshape(T))
    ys = _experts(xe, eg_w, eg_s, eu_w, eu_s, ed_w, ed_s)
    y0, y1 = _combine(ys, dc0.reshape(T), dc1.reshape(T))
    return _shared(x, sg_w, sg_s, su_w, su_s, sd_w, sd_s, y0, y1, g0, g1)
